# baseline (speedup 1.0000x reference)
import functools

import jax
import jax.numpy as jnp
from jax import lax
from jax.experimental import pallas as pl
from jax.experimental.pallas import tpu as pltpu
from jax.experimental.pallas import tpu_sc as plsc

VOCAB = 512
SEQ = 512
HID = 768
BATCH = 256

NC = 2
NS = 16
NW = NC * NS
LANES = 16

S_PER_W = SEQ // NW
KB = 2
CHUNK = KB * S_PER_W
NIT = BATCH // KB
NBUF = 4
NGRP = NIT // NBUF

_mesh = plsc.VectorSubcoreMesh(core_axis_name="c", subcore_axis_name="s")


@functools.partial(
    pl.kernel,
    out_type=jax.ShapeDtypeStruct((BATCH, SEQ, HID), jnp.float32),
    mesh=_mesh,
    scratch_types=[
        pltpu.VMEM((CHUNK, HID), jnp.float32),
        pltpu.VMEM((BATCH * S_PER_W,), jnp.int32),
        [pltpu.VMEM((CHUNK, HID), jnp.float32)] * NBUF,
        [pltpu.SemaphoreType.DMA] * NBUF,
        [pltpu.SemaphoreType.DMA] * NBUF,
    ],
)
def _sc_embed(xt_hbm, table_hbm, pos_hbm, out_hbm, slab, idxv, bufs, gsems, wsems):
    cid = lax.axis_index("c")
    sid = lax.axis_index("s")
    wid = sid * NC + cid
    s0 = wid * S_PER_W

    for k in range(KB):
        pltpu.sync_copy(pos_hbm.at[pl.ds(s0, S_PER_W)], slab.at[pl.ds(k * S_PER_W, S_PER_W)])
    pltpu.sync_copy(xt_hbm.at[wid], idxv)

    def fire_gather(c, j):
        idx = idxv.at[pl.ds(c * CHUNK, CHUNK)]
        pltpu.make_async_copy(table_hbm.at[idx], bufs[j], gsems[j]).start()

    def wait_gather(j):
        idx = idxv.at[pl.ds(0, CHUNK)]
        pltpu.make_async_copy(table_hbm.at[idx], bufs[j], gsems[j]).wait()

    def fire_writes(c, j):
        b0 = c * KB
        for k in range(KB):
            pltpu.make_async_copy(
                bufs[j].at[pl.ds(k * S_PER_W, S_PER_W)],
                out_hbm.at[b0 + k, pl.ds(s0, S_PER_W)],
                wsems[j],
            ).start()

    def wait_writes(j):
        for k in range(KB):
            pltpu.make_async_copy(
                bufs[j].at[pl.ds(k * S_PER_W, S_PER_W)],
                out_hbm.at[k, pl.ds(s0, S_PER_W)],
                wsems[j],
            ).wait()

    def add_slab(j):
        buf = bufs[j]

        def row_body(r, cy):
            def col_body(cc, cy2):
                off = cc * LANES
                plsc.addupdate(buf.at[r, pl.ds(off, LANES)], slab[r, pl.ds(off, LANES)])
                return cy2

            return lax.fori_loop(0, HID // LANES, col_body, cy, unroll=16)

        lax.fori_loop(0, CHUNK, row_body, 0)

    for j in range(NBUF - 1):
        fire_gather(j, j)

    def group_body(h, carry):
        for j in range(NBUF):
            c = h * NBUF + j
            wait_gather(j)
            add_slab(j)
            fire_writes(c, j)
            nb = (j + NBUF - 1) % NBUF
            if j == 0:
                @pl.when(h >= 1)
                def _():
                    wait_writes(nb)

                fire_gather(c + NBUF - 1, nb)
            else:
                wait_writes(nb)

                @pl.when(h < NGRP - 1)
                def _():
                    fire_gather(c + NBUF - 1, nb)
        return carry

    lax.fori_loop(0, NGRP, group_body, 0)

    wait_writes(NBUF - 1)


@jax.jit
def kernel(x, patch_table, position_embeddings):
    xt = (
        x.reshape(BATCH, NW, S_PER_W)
        .transpose(1, 0, 2)
        .reshape(NW, BATCH * S_PER_W)
    )
    pos = position_embeddings.reshape(SEQ, HID)
    return _sc_embed(xt, patch_table, pos)


# speedup vs baseline: 2.3915x; 2.3915x over previous
import functools

import jax
import jax.numpy as jnp
from jax import lax
from jax.experimental import pallas as pl
from jax.experimental.pallas import tpu as pltpu
from jax.experimental.pallas import tpu_sc as plsc

VOCAB = 512
SEQ = 512
HID = 768
BATCH = 256

NC = 2
NS = 16
NW = NC * NS
LANES = 16

S_PER_W = SEQ // NW
KB = 2
CHUNK = KB * S_PER_W
NIT = BATCH // KB
NBUF = 4
NGRP = NIT // NBUF

_mesh = plsc.VectorSubcoreMesh(core_axis_name="c", subcore_axis_name="s")


@functools.partial(
    pl.kernel,
    out_type=jax.ShapeDtypeStruct((BATCH, SEQ, HID), jnp.float32),
    mesh=_mesh,
    scratch_types=[
        pltpu.VMEM((CHUNK, HID), jnp.float32),
        pltpu.VMEM((BATCH * S_PER_W,), jnp.int32),
        [pltpu.VMEM((CHUNK, HID), jnp.float32)] * NBUF,
        [pltpu.SemaphoreType.DMA] * NBUF,
        [pltpu.SemaphoreType.DMA] * NBUF,
    ],
)
def _sc_embed(xt_hbm, table_hbm, pos_hbm, out_hbm, slab, idxv, bufs, gsems, wsems):
    cid = lax.axis_index("c")
    sid = lax.axis_index("s")
    wid = sid * NC + cid
    s0 = wid * S_PER_W

    for k in range(KB):
        pltpu.sync_copy(pos_hbm.at[pl.ds(s0, S_PER_W)], slab.at[pl.ds(k * S_PER_W, S_PER_W)])
    pltpu.sync_copy(xt_hbm.at[wid], idxv)

    def fire_gather(c, j):
        idx = idxv.at[pl.ds(c * CHUNK, CHUNK)]
        pltpu.make_async_copy(table_hbm.at[idx], bufs[j], gsems[j]).start()

    def wait_gather(j):
        idx = idxv.at[pl.ds(0, CHUNK)]
        pltpu.make_async_copy(table_hbm.at[idx], bufs[j], gsems[j]).wait()

    def fire_writes(c, j):
        b0 = c * KB
        for k in range(KB):
            pltpu.make_async_copy(
                bufs[j].at[pl.ds(k * S_PER_W, S_PER_W)],
                out_hbm.at[b0 + k, pl.ds(s0, S_PER_W)],
                wsems[j],
            ).start()

    def wait_writes(j):
        for k in range(KB):
            pltpu.make_async_copy(
                bufs[j].at[pl.ds(k * S_PER_W, S_PER_W)],
                out_hbm.at[k, pl.ds(s0, S_PER_W)],
                wsems[j],
            ).wait()

    def add_slab(j):
        buf = bufs[j]

        @plsc.parallel_loop(0, CHUNK, unroll=2)
        def _(r):
            for cc in range(HID // LANES):
                off = cc * LANES
                plsc.addupdate(buf.at[r, pl.ds(off, LANES)], slab[r, pl.ds(off, LANES)])

    for j in range(NBUF - 1):
        fire_gather(j, j)

    def group_body(h, carry):
        for j in range(NBUF):
            c = h * NBUF + j
            wait_gather(j)
            add_slab(j)
            fire_writes(c, j)
            nb = (j + NBUF - 1) % NBUF
            if j == 0:
                @pl.when(h >= 1)
                def _():
                    wait_writes(nb)

                fire_gather(c + NBUF - 1, nb)
            else:
                wait_writes(nb)

                @pl.when(h < NGRP - 1)
                def _():
                    fire_gather(c + NBUF - 1, nb)
        return carry

    lax.fori_loop(0, NGRP, group_body, 0)

    wait_writes(NBUF - 1)


@jax.jit
def kernel(x, patch_table, position_embeddings):
    xt = (
        x.reshape(BATCH, NW, S_PER_W)
        .transpose(1, 0, 2)
        .reshape(NW, BATCH * S_PER_W)
    )
    pos = position_embeddings.reshape(SEQ, HID)
    return _sc_embed(xt, patch_table, pos)
